# Initial kernel scaffold; baseline (speedup 1.0000x reference)
import jax
import jax.numpy as jnp
from jax.experimental import pallas as pl


def kernel(x, neigh_orders):
    raise NotImplementedError("write your pallas kernel here")


# speedup vs baseline: 2.8712x; 2.8712x over previous
import functools
import jax
import jax.numpy as jnp
from jax import lax
from jax.experimental import pallas as pl
from jax.experimental.pallas import tpu as pltpu
from jax.experimental.pallas import tpu_sc as plsc

N_IN = 163842
D = 256
N_OUT = (N_IN + 6) // 4
K = 7
L = 16
C = 16
NCHUNK = (N_OUT + C - 1) // C
TAIL = N_OUT - (NCHUNK - 1) * C


def _make_kernel():
    info = plsc.get_sparse_core_info()
    NC, NS = info.num_cores, info.num_subcores
    NW = NC * NS
    nt = (NCHUNK + NW - 1) // NW

    mesh = plsc.VectorSubcoreMesh(core_axis_name="c", subcore_axis_name="s")

    @functools.partial(
        pl.kernel,
        mesh=mesh,
        out_type=jax.ShapeDtypeStruct((N_OUT, D), jnp.float32),
        scratch_types=[
            pltpu.VMEM((C * K,), jnp.int32),
            pltpu.VMEM((C * K, D), jnp.float32),
            pltpu.VMEM((C, D), jnp.float32),
            pltpu.SemaphoreType.DMA,
            pltpu.SemaphoreType.DMA,
        ],
    )
    def pool_kernel(x_hbm, idx_hbm, out_hbm, idx_v, rows_v, out_v, gsem, osem):
        wid = lax.axis_index("s") * NC + lax.axis_index("c")

        def step(t, carry):
            g = t * NW + wid

            @pl.when(g < NCHUNK)
            def _():
                base = g * C
                pltpu.sync_copy(idx_hbm.at[pl.ds(g * (C * K), C * K)], idx_v)
                pltpu.async_copy(x_hbm.at[idx_v], rows_v, gsem).wait()

                def comp(i, c2):
                    for s in range(D // L):
                        acc = rows_v[i * K, pl.ds(s * L, L)]
                        for j in range(1, K):
                            acc = acc + rows_v[i * K + j, pl.ds(s * L, L)]
                        out_v[i, pl.ds(s * L, L)] = acc * jnp.float32(1.0 / K)
                    return c2

                lax.fori_loop(0, C, comp, 0, unroll=True)

                @pl.when(g < NCHUNK - 1)
                def _():
                    pltpu.async_copy(out_v, out_hbm.at[pl.ds(base, C)], osem).wait()

                @pl.when(g == NCHUNK - 1)
                def _():
                    pltpu.async_copy(
                        out_v.at[pl.ds(0, TAIL)],
                        out_hbm.at[pl.ds(base, TAIL)],
                        osem,
                    ).wait()

            return carry

        lax.fori_loop(0, nt, step, 0)

    return pool_kernel


_POOL_KERNEL = _make_kernel()


@jax.jit
def kernel(x, neigh_orders):
    idx = neigh_orders[: N_OUT * K]
    pad = NCHUNK * C * K - N_OUT * K
    idx = jnp.concatenate([idx, jnp.zeros((pad,), jnp.int32)])
    return _POOL_KERNEL(x, idx)


# speedup vs baseline: 7.0712x; 2.4628x over previous
import functools
import jax
import jax.numpy as jnp
from jax import lax
from jax.experimental import pallas as pl
from jax.experimental.pallas import tpu as pltpu
from jax.experimental.pallas import tpu_sc as plsc

N_IN = 163842
D = 256
N_OUT = (N_IN + 6) // 4
K = 7
L = 16
C = 16
NCHUNK = (N_OUT + C - 1) // C
TAIL = N_OUT - (NCHUNK - 1) * C
NBUF = 3


def _make_kernel():
    info = plsc.get_sparse_core_info()
    NC, NS = info.num_cores, info.num_subcores
    NW = NC * NS
    CPW = -(-NCHUNK // NW)
    assert CPW % NBUF == 0
    n_pad_chunks = NW * CPW

    mesh = plsc.VectorSubcoreMesh(core_axis_name="c", subcore_axis_name="s")

    @functools.partial(
        pl.kernel,
        mesh=mesh,
        out_type=jax.ShapeDtypeStruct((N_OUT, D), jnp.float32),
        scratch_types=[
            pltpu.VMEM((CPW, C * K), jnp.int32),
            pltpu.VMEM((NBUF, C * K, D), jnp.float32),
            pltpu.VMEM((NBUF, C, D), jnp.float32),
            pltpu.SemaphoreType.DMA,
            pltpu.SemaphoreType.DMA,
            pltpu.SemaphoreType.DMA,
            pltpu.SemaphoreType.DMA,
            pltpu.SemaphoreType.DMA,
            pltpu.SemaphoreType.DMA,
        ],
    )
    def pool_kernel(x_hbm, idx_hbm, out_hbm, idx_v, rows_v, out_v,
                    g0, g1, g2, o0, o1, o2):
        gsem = [g0, g1, g2]
        osem = [o0, o1, o2]
        wid = lax.axis_index("s") * NC + lax.axis_index("c")
        first = wid * CPW

        pltpu.sync_copy(idx_hbm.at[wid], idx_v)

        def fire_gather(b, t):
            pltpu.async_copy(x_hbm.at[idx_v.at[t]], rows_v.at[b], gsem[b])

        def wait_gather(b):
            pltpu.make_async_copy(
                x_hbm.at[idx_v.at[0]], rows_v.at[b], gsem[b]).wait()

        def fire_out(b, g):
            base = g * C

            @pl.when(g < NCHUNK - 1)
            def _():
                pltpu.async_copy(out_v.at[b], out_hbm.at[pl.ds(base, C)],
                                 osem[b])

            @pl.when(g == NCHUNK - 1)
            def _():
                pltpu.async_copy(out_v.at[b, pl.ds(0, TAIL)],
                                 out_hbm.at[pl.ds(base, TAIL)], osem[b])

        def drain_out(b, g_prev):
            @pl.when(g_prev < NCHUNK - 1)
            def _():
                pltpu.make_async_copy(
                    out_v.at[b], out_hbm.at[pl.ds(0, C)], osem[b]).wait()

            @pl.when(g_prev == NCHUNK - 1)
            def _():
                pltpu.make_async_copy(
                    out_v.at[b, pl.ds(0, TAIL)],
                    out_hbm.at[pl.ds(0, TAIL)], osem[b]).wait()

        def compute(b, i, _):
            for s in range(D // L):
                v = [rows_v[b, i * K + j, pl.ds(s * L, L)] for j in range(K)]
                t0 = v[0] + v[1]
                t1 = v[2] + v[3]
                t2 = v[4] + v[5]
                acc = (t0 + t1) + (t2 + v[6])
                out_v[b, i, pl.ds(s * L, L)] = acc * jnp.float32(1.0 / K)
            return _

        for b in range(NBUF):
            fire_gather(b, b)

        def step(tt, carry):
            for b in range(NBUF):
                t = tt * NBUF + b
                g = first + t

                @pl.when(t - NBUF >= 0)
                def _():
                    drain_out(b, g - NBUF)

                @pl.when(g < NCHUNK)
                def _():
                    wait_gather(b)
                    lax.fori_loop(0, C, functools.partial(compute, b), 0)
                    fire_out(b, g)

                @pl.when((t + NBUF < CPW) & (g + NBUF < NCHUNK))
                def _():
                    fire_gather(b, t + NBUF)
            return carry

        lax.fori_loop(0, CPW // NBUF, step, 0)

        for b in range(NBUF):
            t = CPW + b
            g_prev = first + t - NBUF

            @pl.when(g_prev < NCHUNK)
            def _():
                drain_out(b, g_prev)

    return pool_kernel


_POOL_KERNEL = _make_kernel()


@jax.jit
def kernel(x, neigh_orders):
    info = plsc.get_sparse_core_info()
    nw = info.num_cores * info.num_subcores
    cpw = -(-NCHUNK // nw)
    idx = neigh_orders[: N_OUT * K]
    pad = nw * cpw * C * K - N_OUT * K
    idx = jnp.concatenate([idx, jnp.zeros((pad,), jnp.int32)])
    return _POOL_KERNEL(x, idx.reshape(nw, cpw, C * K))


# speedup vs baseline: 9.8228x; 1.3891x over previous
import functools
import jax
import jax.numpy as jnp
from jax import lax
from jax.experimental import pallas as pl
from jax.experimental.pallas import tpu as pltpu
from jax.experimental.pallas import tpu_sc as plsc

N_IN = 163842
D = 256
N_OUT = (N_IN + 6) // 4
K = 7
L = 16
C = 16
NCHUNK = (N_OUT + C - 1) // C
TAIL = N_OUT - (NCHUNK - 1) * C
NBUF = 3


def _make_kernel():
    info = plsc.get_sparse_core_info()
    NC, NS = info.num_cores, info.num_subcores
    NW = NC * NS
    CPW = -(-NCHUNK // NW)
    assert CPW % NBUF == 0
    n_pad_chunks = NW * CPW

    mesh = plsc.VectorSubcoreMesh(core_axis_name="c", subcore_axis_name="s")

    @functools.partial(
        pl.kernel,
        mesh=mesh,
        out_type=jax.ShapeDtypeStruct((N_OUT, D), jnp.float32),
        scratch_types=[
            pltpu.VMEM((CPW, C * K), jnp.int32),
            pltpu.VMEM((NBUF, C * K, D), jnp.float32),
            pltpu.VMEM((NBUF, C, D), jnp.float32),
            pltpu.SemaphoreType.DMA,
            pltpu.SemaphoreType.DMA,
            pltpu.SemaphoreType.DMA,
            pltpu.SemaphoreType.DMA,
            pltpu.SemaphoreType.DMA,
            pltpu.SemaphoreType.DMA,
        ],
    )
    def pool_kernel(x_hbm, idx_hbm, out_hbm, idx_v, rows_v, out_v,
                    g0, g1, g2, o0, o1, o2):
        gsem = [g0, g1, g2]
        osem = [o0, o1, o2]
        wid = lax.axis_index("s") * NC + lax.axis_index("c")
        first = wid * CPW

        pltpu.sync_copy(idx_hbm.at[wid], idx_v)

        def fire_gather(b, t):
            pltpu.async_copy(x_hbm.at[idx_v.at[t]], rows_v.at[b], gsem[b])

        def wait_gather(b):
            pltpu.make_async_copy(
                x_hbm.at[idx_v.at[0]], rows_v.at[b], gsem[b]).wait()

        def fire_out(b, g):
            base = g * C

            @pl.when(g < NCHUNK - 1)
            def _():
                pltpu.async_copy(out_v.at[b], out_hbm.at[pl.ds(base, C)],
                                 osem[b])

            @pl.when(g == NCHUNK - 1)
            def _():
                pltpu.async_copy(out_v.at[b, pl.ds(0, TAIL)],
                                 out_hbm.at[pl.ds(base, TAIL)], osem[b])

        def drain_out(b, g_prev):
            @pl.when(g_prev < NCHUNK - 1)
            def _():
                pltpu.make_async_copy(
                    out_v.at[b], out_hbm.at[pl.ds(0, C)], osem[b]).wait()

            @pl.when(g_prev == NCHUNK - 1)
            def _():
                pltpu.make_async_copy(
                    out_v.at[b, pl.ds(0, TAIL)],
                    out_hbm.at[pl.ds(0, TAIL)], osem[b]).wait()

        def compute(b, i, _):
            for s in range(0, D // L, 2):
                va = [rows_v[b, i * K + j, pl.ds(s * L, L)] for j in range(K)]
                vb = [rows_v[b, i * K + j, pl.ds((s + 1) * L, L)]
                      for j in range(K)]
                aa = ((va[0] + va[1]) + (va[2] + va[3])) + \
                     ((va[4] + va[5]) + va[6])
                ab = ((vb[0] + vb[1]) + (vb[2] + vb[3])) + \
                     ((vb[4] + vb[5]) + vb[6])
                out_v[b, i, pl.ds(s * L, L)] = aa * jnp.float32(1.0 / K)
                out_v[b, i, pl.ds((s + 1) * L, L)] = ab * jnp.float32(1.0 / K)
            return _

        for b in range(NBUF):
            fire_gather(b, b)

        def step(tt, carry):
            for b in range(NBUF):
                t = tt * NBUF + b
                g = first + t

                @pl.when(t - NBUF >= 0)
                def _():
                    drain_out(b, g - NBUF)

                @pl.when(g < NCHUNK)
                def _():
                    wait_gather(b)
                    lax.fori_loop(0, C, functools.partial(compute, b), 0)
                    fire_out(b, g)

                @pl.when((t + NBUF < CPW) & (g + NBUF < NCHUNK))
                def _():
                    fire_gather(b, t + NBUF)
            return carry

        lax.fori_loop(0, CPW // NBUF, step, 0)

        for b in range(NBUF):
            t = CPW + b
            g_prev = first + t - NBUF

            @pl.when(g_prev < NCHUNK)
            def _():
                drain_out(b, g_prev)

    return pool_kernel


_POOL_KERNEL = _make_kernel()


@jax.jit
def kernel(x, neigh_orders):
    info = plsc.get_sparse_core_info()
    nw = info.num_cores * info.num_subcores
    cpw = -(-NCHUNK // nw)
    idx = neigh_orders[: N_OUT * K]
    pad = nw * cpw * C * K - N_OUT * K
    idx = jnp.concatenate([idx, jnp.zeros((pad,), jnp.int32)])
    return _POOL_KERNEL(x, idx.reshape(nw, cpw, C * K))


# speedup vs baseline: 11.3918x; 1.1597x over previous
import functools
import jax
import jax.numpy as jnp
from jax import lax
from jax.experimental import pallas as pl
from jax.experimental.pallas import tpu as pltpu
from jax.experimental.pallas import tpu_sc as plsc

N_IN = 163842
D = 256
N_OUT = (N_IN + 6) // 4
K = 7
L = 16
C = 16
NCHUNK = (N_OUT + C - 1) // C
TAIL = N_OUT - (NCHUNK - 1) * C
NBUF = 3


def _make_kernel():
    info = plsc.get_sparse_core_info()
    NC, NS = info.num_cores, info.num_subcores
    NW = NC * NS
    CPW = -(-NCHUNK // NW)
    assert CPW % NBUF == 0
    n_pad_chunks = NW * CPW

    mesh = plsc.VectorSubcoreMesh(core_axis_name="c", subcore_axis_name="s")

    @functools.partial(
        pl.kernel,
        mesh=mesh,
        out_type=jax.ShapeDtypeStruct((N_OUT, D), jnp.float32),
        scratch_types=[
            pltpu.VMEM((CPW, C * K), jnp.int32),
            pltpu.VMEM((NBUF, C * K, D), jnp.float32),
            pltpu.VMEM((NBUF, C, D), jnp.float32),
            pltpu.SemaphoreType.DMA,
            pltpu.SemaphoreType.DMA,
            pltpu.SemaphoreType.DMA,
            pltpu.SemaphoreType.DMA,
            pltpu.SemaphoreType.DMA,
            pltpu.SemaphoreType.DMA,
        ],
    )
    def pool_kernel(x_hbm, idx_hbm, out_hbm, idx_v, rows_v, out_v,
                    g0, g1, g2, o0, o1, o2):
        gsem = [g0, g1, g2]
        osem = [o0, o1, o2]
        wid = lax.axis_index("s") * NC + lax.axis_index("c")
        first = wid * CPW

        pltpu.sync_copy(idx_hbm.at[wid], idx_v)

        def fire_gather(b, t):
            pltpu.async_copy(x_hbm.at[idx_v.at[t]], rows_v.at[b], gsem[b])

        def wait_gather(b):
            pltpu.make_async_copy(
                x_hbm.at[idx_v.at[0]], rows_v.at[b], gsem[b]).wait()

        def fire_out(b, g):
            base = g * C

            @pl.when(g < NCHUNK - 1)
            def _():
                pltpu.async_copy(out_v.at[b], out_hbm.at[pl.ds(base, C)],
                                 osem[b])

            @pl.when(g == NCHUNK - 1)
            def _():
                pltpu.async_copy(out_v.at[b, pl.ds(0, TAIL)],
                                 out_hbm.at[pl.ds(base, TAIL)], osem[b])

        def drain_out(b, g_prev):
            @pl.when(g_prev < NCHUNK - 1)
            def _():
                pltpu.make_async_copy(
                    out_v.at[b], out_hbm.at[pl.ds(0, C)], osem[b]).wait()

            @pl.when(g_prev == NCHUNK - 1)
            def _():
                pltpu.make_async_copy(
                    out_v.at[b, pl.ds(0, TAIL)],
                    out_hbm.at[pl.ds(0, TAIL)], osem[b]).wait()

        def compute(b, i, _):
            for s in range(0, D // L, 4):
                vs = [[rows_v[b, i * K + j, pl.ds((s + q) * L, L)]
                       for j in range(K)] for q in range(4)]
                for q in range(4):
                    v = vs[q]
                    acc = ((v[0] + v[1]) + (v[2] + v[3])) + \
                          ((v[4] + v[5]) + v[6])
                    out_v[b, i, pl.ds((s + q) * L, L)] = \
                        acc * jnp.float32(1.0 / K)
            return _

        for b in range(NBUF):
            fire_gather(b, b)

        def step(tt, carry):
            for b in range(NBUF):
                t = tt * NBUF + b
                g = first + t

                @pl.when(t - NBUF >= 0)
                def _():
                    drain_out(b, g - NBUF)

                @pl.when(g < NCHUNK)
                def _():
                    wait_gather(b)
                    lax.fori_loop(0, C, functools.partial(compute, b), 0)
                    fire_out(b, g)

                @pl.when((t + NBUF < CPW) & (g + NBUF < NCHUNK))
                def _():
                    fire_gather(b, t + NBUF)
            return carry

        lax.fori_loop(0, CPW // NBUF, step, 0)

        for b in range(NBUF):
            t = CPW + b
            g_prev = first + t - NBUF

            @pl.when(g_prev < NCHUNK)
            def _():
                drain_out(b, g_prev)

    return pool_kernel


_POOL_KERNEL = _make_kernel()


@jax.jit
def kernel(x, neigh_orders):
    info = plsc.get_sparse_core_info()
    nw = info.num_cores * info.num_subcores
    cpw = -(-NCHUNK // nw)
    idx = neigh_orders[: N_OUT * K]
    pad = nw * cpw * C * K - N_OUT * K
    idx = jnp.concatenate([idx, jnp.zeros((pad,), jnp.int32)])
    return _POOL_KERNEL(x, idx.reshape(nw, cpw, C * K))
